# baseline (speedup 1.0000x reference)
import jax
import jax.numpy as jnp
from jax import lax
from jax.experimental import pallas as pl
from jax.experimental.pallas import tpu as pltpu
from jax.experimental.pallas import tpu_sc as plsc

_LAMB = 1.0
_N = 10000
_E = 320000
_D = 128
_NC = 2
_NS = 16
_NW = _NC * _NS
_EPW = _E // _NW
_K = 128
_KR = 125
_NCH = _EPW // _KR
_NP = 10240
_RPT = _NP // _NS


def _deg_body(row_hbm, col_hbm, degp_hbm, diagp_hbm, rowv, colv, hist, hist2):
    cid = lax.axis_index("c")
    sid = lax.axis_index("s")
    wid = cid * _NS + sid
    pltpu.sync_copy(row_hbm.at[wid], rowv)
    pltpu.sync_copy(col_hbm.at[wid], colv)
    zeros16 = jnp.zeros((16,), jnp.float32)

    def zloop(i, c):
        hist[pl.ds(i * 16, 16)] = zeros16
        hist2[pl.ds(i * 16, 16)] = zeros16
        return c

    lax.fori_loop(0, _N // 16, zloop, 0)
    ones16 = jnp.ones((16,), jnp.float32)

    def step(i, c):
        c16 = colv[pl.ds(i * 16, 16)]
        plsc.addupdate_scatter(hist, [c16], ones16)
        r16 = rowv[pl.ds(i * 16, 16)]
        plsc.addupdate_scatter(hist2, [r16], ones16, mask=r16 == c16)
        return c

    lax.fori_loop(0, _EPW // 16, step, 0)
    pltpu.sync_copy(hist, degp_hbm.at[wid])
    pltpu.sync_copy(hist2, diagp_hbm.at[wid])


_deg = pl.kernel(
    _deg_body,
    out_type=(
        jax.ShapeDtypeStruct((_NW, _N), jnp.float32),
        jax.ShapeDtypeStruct((_NW, _N), jnp.float32),
    ),
    mesh=plsc.VectorSubcoreMesh(core_axis_name="c", subcore_axis_name="s"),
    compiler_params=pltpu.CompilerParams(needs_layout_passes=False),
    scratch_types=[
        pltpu.VMEM((_EPW,), jnp.int32),
        pltpu.VMEM((_EPW,), jnp.int32),
        pltpu.VMEM((_N,), jnp.float32),
        pltpu.VMEM((_N,), jnp.float32),
    ],
)


_NB = 2


def _agg_body(z_hbm, rowp_hbm, colp_hbm, zeros_hbm, parts_hbm,
              colv, ir0, ir1, b0, b1, acc,
              gA, gB, sA, sB, irA, irB):
    irs = (ir0, ir1)
    bufs = (b0, b1)
    gs = (gA, gB)
    ss = (sA, sB)
    irsem = (irA, irB)
    cid = lax.axis_index("c")
    sid = lax.axis_index("s")
    wid = cid * _NS + sid
    pltpu.sync_copy(zeros_hbm.at[pl.ds(sid * _RPT, _RPT)],
                    acc.at[pl.ds(sid * _RPT, _RPT)])
    pltpu.sync_copy(colp_hbm.at[wid], colv)
    plsc.subcore_barrier()
    rbase = wid * _NCH * _K

    def ir_start(j, p):
        pltpu.async_copy(rowp_hbm.at[pl.ds(rbase + j * _K, _K)], irs[p],
                         irsem[p])

    def ir_wait(p):
        pltpu.make_async_copy(rowp_hbm.at[pl.ds(rbase, _K)], irs[p],
                              irsem[p]).wait()

    def g_start(j, p):
        pltpu.async_copy(z_hbm.at[colv.at[j]], bufs[p], gs[p])

    def g_wait(p):
        pltpu.make_async_copy(z_hbm.at[colv.at[0]], bufs[p], gs[p]).wait()

    def s_start(p):
        pltpu.async_copy(bufs[p], acc.at[irs[p]], ss[p], add=True)

    def s_wait(p):
        pltpu.make_async_copy(bufs[p], acc.at[irs[p]], ss[p]).wait()

    ir_start(0, 0)
    g_start(0, 0)
    ng = (_NCH + 2) // 2

    def group(i, carry):
        jb = i * 2
        for b in range(2):
            j = jb + b
            pg = (b + 1) % 2

            @pl.when((j >= 1) & (j <= _NCH))
            def _():
                s_wait(pg)

            ir_start(jnp.minimum(j + 1, _NCH - 1), pg)
            g_start(jnp.minimum(j + 1, _NCH - 1), pg)
            g_wait(b)
            ir_wait(b)

            @pl.when(j < _NCH)
            def _():
                s_start(b)

        return carry

    lax.fori_loop(0, ng, group, 0)
    ir_wait(0)
    g_wait(0)
    plsc.subcore_barrier()
    pltpu.sync_copy(acc.at[pl.ds(sid * _RPT, _RPT)],
                    parts_hbm.at[cid, pl.ds(sid * _RPT, _RPT)])


_agg = pl.kernel(
    _agg_body,
    out_type=jax.ShapeDtypeStruct((_NC, _NP, _D), jnp.float32),
    mesh=plsc.VectorSubcoreMesh(core_axis_name="c", subcore_axis_name="s"),
    scratch_types=(
        [pltpu.VMEM((_NCH, _K), jnp.int32)]
        + [pltpu.VMEM((_K,), jnp.int32) for _ in range(2)]
        + [pltpu.VMEM((_K, _D), jnp.float32) for _ in range(2)]
        + [pltpu.VMEM_SHARED((_NP, _D), jnp.float32)]
        + [pltpu.SemaphoreType.DMA for _ in range(6)]
    ),
)


def _prep_body(degp_ref, diagp_ref, x_ref, z_ref, dn_ref, cv_ref):
    deg = 1.0 + jnp.sum(degp_ref[...], axis=0)
    dn = 1.0 / deg
    cv = dn + _LAMB * (1.0 + jnp.sum(diagp_ref[...], axis=0))
    dn_ref[...] = dn[:, None]
    cv_ref[...] = cv[:, None]
    z_ref[...] = dn[:, None] * x_ref[...]


def _prep(degp, diagp, x):
    return pl.pallas_call(
        _prep_body,
        out_shape=(
            jax.ShapeDtypeStruct((_N, _D), jnp.float32),
            jax.ShapeDtypeStruct((_N, 1), jnp.float32),
            jax.ShapeDtypeStruct((_N, 1), jnp.float32),
        ),
    )(degp, diagp, x)


def _layer1_body(p_ref, x_ref, cv_ref, dn_ref, w_ref, b_ref, y_ref, z_ref):
    u = p_ref[0] + p_ref[1] + cv_ref[...] * x_ref[...]
    y = jnp.maximum(u @ w_ref[...] + b_ref[...][None, :], 0.0)
    y_ref[...] = y
    z_ref[...] = dn_ref[...] * y


def _layer1(p, x, cv, dn, W1, b1):
    BR = 1000
    return pl.pallas_call(
        _layer1_body,
        grid=(_N // BR,),
        in_specs=[
            pl.BlockSpec((_NC, BR, _D), lambda i: (0, i, 0)),
            pl.BlockSpec((BR, _D), lambda i: (i, 0)),
            pl.BlockSpec((BR, 1), lambda i: (i, 0)),
            pl.BlockSpec((BR, 1), lambda i: (i, 0)),
            pl.BlockSpec((_D, _D), lambda i: (0, 0)),
            pl.BlockSpec((_D,), lambda i: (0,)),
        ],
        out_specs=(
            pl.BlockSpec((BR, _D), lambda i: (i, 0)),
            pl.BlockSpec((BR, _D), lambda i: (i, 0)),
        ),
        out_shape=(
            jax.ShapeDtypeStruct((_N, _D), jnp.float32),
            jax.ShapeDtypeStruct((_N, _D), jnp.float32),
        ),
    )(p, x, cv, dn, W1, b1)


def _layer2_body(p_ref, y_ref, cv_ref, w_ref, b_ref, o_ref):
    u = p_ref[0] + p_ref[1] + cv_ref[...] * y_ref[...]
    v = u @ w_ref[...] + b_ref[...][None, :]
    m = jnp.max(v, axis=-1, keepdims=True)
    e = jnp.exp(v - m)
    o_ref[...] = v - m - jnp.log(jnp.sum(e, axis=-1, keepdims=True))


def _layer2(p, y1, cv, W2, b2):
    BR = 1000
    return pl.pallas_call(
        _layer2_body,
        grid=(_N // BR,),
        in_specs=[
            pl.BlockSpec((_NC, BR, _D), lambda i: (0, i, 0)),
            pl.BlockSpec((BR, _D), lambda i: (i, 0)),
            pl.BlockSpec((BR, 1), lambda i: (i, 0)),
            pl.BlockSpec((_D, _D), lambda i: (0, 0)),
            pl.BlockSpec((_D,), lambda i: (0,)),
        ],
        out_specs=pl.BlockSpec((BR, _D), lambda i: (i, 0)),
        out_shape=jax.ShapeDtypeStruct((_N, _D), jnp.float32),
    )(p, y1, cv, W2, b2)


def kernel(x, edge_index, W1, b1, W2, b2):
    row = edge_index[0].astype(jnp.int32)
    col = edge_index[1].astype(jnp.int32)
    rowf = row.reshape(_NW, _EPW)
    colf = col.reshape(_NW, _EPW)
    rowp = jnp.pad(row.reshape(_NW, _NCH, _KR), ((0, 0), (0, 0), (0, 3)),
                   constant_values=_N).reshape(-1)
    colp = jnp.pad(col.reshape(_NW, _NCH, _KR), ((0, 0), (0, 0), (0, 3)),
                   constant_values=0)
    zeros = jnp.zeros((_NP, _D), jnp.float32)

    degp, diagp = _deg(rowf, colf)
    z1, dn, cv = _prep(degp, diagp, x)
    p1 = _agg(z1, rowp, colp, zeros)
    y1, z2 = _layer1(p1[:, :_N], x, cv, dn, W1, b1)
    p2 = _agg(z2, rowp, colp, zeros)
    return _layer2(p2[:, :_N], y1, cv, W2, b2)


# speedup vs baseline: 2.7453x; 2.7453x over previous
import jax
import jax.numpy as jnp
from jax import lax
from jax.experimental import pallas as pl
from jax.experimental.pallas import tpu as pltpu
from jax.experimental.pallas import tpu_sc as plsc

_LAMB = 1.0
_N = 10000
_E = 320000
_D = 128
_NC = 2
_NS = 16
_NW = _NC * _NS
_EPW = _E // _NW
_K = 128
_KR = 125
_NCH = _EPW // _KR
_NP = 10240
_RPT = _NP // _NS


def _deg_body(row_hbm, col_hbm, degp_hbm, diagp_hbm, rowv, colv, hist, hist2):
    cid = lax.axis_index("c")
    sid = lax.axis_index("s")
    wid = cid * _NS + sid
    pltpu.sync_copy(row_hbm.at[wid], rowv)
    pltpu.sync_copy(col_hbm.at[wid], colv)
    zeros16 = jnp.zeros((16,), jnp.float32)

    def zloop(i, c):
        hist[pl.ds(i * 16, 16)] = zeros16
        hist2[pl.ds(i * 16, 16)] = zeros16
        return c

    lax.fori_loop(0, _N // 16, zloop, 0)
    ones16 = jnp.ones((16,), jnp.float32)

    def step(i, c):
        c16 = colv[pl.ds(i * 16, 16)]
        plsc.addupdate_scatter(hist, [c16], ones16)
        r16 = rowv[pl.ds(i * 16, 16)]
        plsc.addupdate_scatter(hist2, [r16], ones16, mask=r16 == c16)
        return c

    lax.fori_loop(0, _EPW // 16, step, 0)
    pltpu.sync_copy(hist, degp_hbm.at[wid])
    pltpu.sync_copy(hist2, diagp_hbm.at[wid])


_deg = pl.kernel(
    _deg_body,
    out_type=(
        jax.ShapeDtypeStruct((_NW, _N), jnp.float32),
        jax.ShapeDtypeStruct((_NW, _N), jnp.float32),
    ),
    mesh=plsc.VectorSubcoreMesh(core_axis_name="c", subcore_axis_name="s"),
    compiler_params=pltpu.CompilerParams(needs_layout_passes=False),
    scratch_types=[
        pltpu.VMEM((_EPW,), jnp.int32),
        pltpu.VMEM((_EPW,), jnp.int32),
        pltpu.VMEM((_N,), jnp.float32),
        pltpu.VMEM((_N,), jnp.float32),
    ],
)


_NB = 2


def _agg_body(z_hbm, rowp_hbm, colp_hbm, zeros_hbm, parts_hbm,
              colv, ir0, ir1, b0, b1, acc,
              gA, gB, sA, sB, irA, irB):
    irs = (ir0, ir1)
    bufs = (b0, b1)
    gs = (gA, gB)
    ss = (sA, sB)
    irsem = (irA, irB)
    cid = lax.axis_index("c")
    sid = lax.axis_index("s")
    wid = cid * _NS + sid
    pltpu.sync_copy(zeros_hbm.at[pl.ds(sid * _RPT, _RPT)],
                    acc.at[pl.ds(sid * _RPT, _RPT)])
    pltpu.sync_copy(colp_hbm.at[wid], colv)
    plsc.subcore_barrier()
    rbase = wid * _NCH * _K

    def ir_start(j, p):
        pltpu.async_copy(rowp_hbm.at[pl.ds(rbase + j * _K, _K)], irs[p],
                         irsem[p])

    def ir_wait(p):
        pltpu.make_async_copy(rowp_hbm.at[pl.ds(rbase, _K)], irs[p],
                              irsem[p]).wait()

    def g_start(j, p):
        pltpu.async_copy(z_hbm.at[colv.at[j]], bufs[p], gs[p])

    def g_wait(p):
        pltpu.make_async_copy(z_hbm.at[colv.at[0]], bufs[p], gs[p]).wait()

    def s_start(p):
        pltpu.async_copy(bufs[p], acc.at[irs[p]], ss[p], add=True)

    def s_wait(p):
        pltpu.make_async_copy(bufs[p], acc.at[irs[p]], ss[p]).wait()

    ir_start(0, 0)
    g_start(0, 0)
    ng = (_NCH + 2) // 2

    def group(i, carry):
        jb = i * 2
        for b in range(2):
            j = jb + b
            pg = (b + 1) % 2

            @pl.when((j >= 1) & (j <= _NCH))
            def _():
                s_wait(pg)

            ir_start(jnp.minimum(j + 1, _NCH - 1), pg)
            g_start(jnp.minimum(j + 1, _NCH - 1), pg)
            g_wait(b)
            ir_wait(b)

            @pl.when(j < _NCH)
            def _():
                s_start(b)

        return carry

    lax.fori_loop(0, ng, group, 0)
    ir_wait(0)
    g_wait(0)
    plsc.subcore_barrier()
    pltpu.sync_copy(acc.at[pl.ds(sid * _RPT, _RPT)],
                    parts_hbm.at[cid, pl.ds(sid * _RPT, _RPT)])


_agg = pl.kernel(
    _agg_body,
    out_type=jax.ShapeDtypeStruct((_NC, _NP, _D), jnp.float32),
    mesh=plsc.VectorSubcoreMesh(core_axis_name="c", subcore_axis_name="s"),
    scratch_types=(
        [pltpu.VMEM((_NCH, _K), jnp.int32)]
        + [pltpu.VMEM((_K,), jnp.int32) for _ in range(2)]
        + [pltpu.VMEM((_K, _D), jnp.float32) for _ in range(2)]
        + [pltpu.VMEM_SHARED((_NP, _D), jnp.float32)]
        + [pltpu.SemaphoreType.DMA for _ in range(6)]
    ),
)


def _prep_body(degp_ref, diagp_ref, x_ref, z_ref, dn_ref, cv_ref):
    deg = 1.0 + jnp.sum(degp_ref[...], axis=0)
    dn = 1.0 / deg
    cv = dn + _LAMB * (1.0 + jnp.sum(diagp_ref[...], axis=0))
    dn_ref[...] = dn[:, None]
    cv_ref[...] = cv[:, None]
    z_ref[...] = dn[:, None] * x_ref[...]


def _prep(degp, diagp, x):
    return pl.pallas_call(
        _prep_body,
        out_shape=(
            jax.ShapeDtypeStruct((_N, _D), jnp.float32),
            jax.ShapeDtypeStruct((_N, 1), jnp.float32),
            jax.ShapeDtypeStruct((_N, 1), jnp.float32),
        ),
    )(degp, diagp, x)


def _layer1_body(p_ref, x_ref, cv_ref, dn_ref, w_ref, b_ref, y_ref, z_ref):
    u = p_ref[0] + p_ref[1] + cv_ref[...] * x_ref[...]
    y = jnp.maximum(u @ w_ref[...] + b_ref[...][None, :], 0.0)
    y_ref[...] = y
    z_ref[...] = dn_ref[...] * y


def _layer1(p, x, cv, dn, W1, b1):
    BR = 1000
    return pl.pallas_call(
        _layer1_body,
        grid=(_N // BR,),
        in_specs=[
            pl.BlockSpec((_NC, BR, _D), lambda i: (0, i, 0)),
            pl.BlockSpec((BR, _D), lambda i: (i, 0)),
            pl.BlockSpec((BR, 1), lambda i: (i, 0)),
            pl.BlockSpec((BR, 1), lambda i: (i, 0)),
            pl.BlockSpec((_D, _D), lambda i: (0, 0)),
            pl.BlockSpec((_D,), lambda i: (0,)),
        ],
        out_specs=(
            pl.BlockSpec((BR, _D), lambda i: (i, 0)),
            pl.BlockSpec((BR, _D), lambda i: (i, 0)),
        ),
        out_shape=(
            jax.ShapeDtypeStruct((_N, _D), jnp.float32),
            jax.ShapeDtypeStruct((_N, _D), jnp.float32),
        ),
    )(p, x, cv, dn, W1, b1)


def _layer2_body(p_ref, y_ref, cv_ref, w_ref, b_ref, o_ref):
    u = p_ref[0] + p_ref[1] + cv_ref[...] * y_ref[...]
    v = u @ w_ref[...] + b_ref[...][None, :]
    m = jnp.max(v, axis=-1, keepdims=True)
    e = jnp.exp(v - m)
    o_ref[...] = v - m - jnp.log(jnp.sum(e, axis=-1, keepdims=True))


def _layer2(p, y1, cv, W2, b2):
    BR = 1000
    return pl.pallas_call(
        _layer2_body,
        grid=(_N // BR,),
        in_specs=[
            pl.BlockSpec((_NC, BR, _D), lambda i: (0, i, 0)),
            pl.BlockSpec((BR, _D), lambda i: (i, 0)),
            pl.BlockSpec((BR, 1), lambda i: (i, 0)),
            pl.BlockSpec((_D, _D), lambda i: (0, 0)),
            pl.BlockSpec((_D,), lambda i: (0,)),
        ],
        out_specs=pl.BlockSpec((BR, _D), lambda i: (i, 0)),
        out_shape=jax.ShapeDtypeStruct((_N, _D), jnp.float32),
    )(p, y1, cv, W2, b2)


def kernel(x, edge_index, W1, b1, W2, b2):
    row = edge_index[0].astype(jnp.int32)
    col = edge_index[1].astype(jnp.int32)
    rowf = row.reshape(_NW, _EPW)
    colf = col.reshape(_NW, _EPW)
    pad_rows = (_N + jnp.arange(_NW * _NCH * 3, dtype=jnp.int32)
                % (_NP - _N)).reshape(_NW, _NCH, 3)
    pad_cols = (jnp.arange(_NW * _NCH * 3, dtype=jnp.int32)
                % _N).reshape(_NW, _NCH, 3)
    rowp = jnp.concatenate([row.reshape(_NW, _NCH, _KR), pad_rows],
                           axis=2).reshape(-1)
    colp = jnp.concatenate([col.reshape(_NW, _NCH, _KR), pad_cols], axis=2)
    zeros = jnp.zeros((_NP, _D), jnp.float32)

    degp, diagp = _deg(rowf, colf)
    z1, dn, cv = _prep(degp, diagp, x)
    p1 = _agg(z1, rowp, colp, zeros)
    y1, z2 = _layer1(p1[:, :_N], x, cv, dn, W1, b1)
    p2 = _agg(z2, rowp, colp, zeros)
    return _layer2(p2[:, :_N], y1, cv, W2, b2)


# speedup vs baseline: 2.9784x; 1.0849x over previous
import jax
import jax.numpy as jnp
from jax import lax
from jax.experimental import pallas as pl
from jax.experimental.pallas import tpu as pltpu
from jax.experimental.pallas import tpu_sc as plsc

_LAMB = 1.0
_N = 10000
_E = 320000
_D = 128
_NC = 2
_NS = 16
_NW = _NC * _NS
_EPW = _E // _NW
_K = 80
_NCH = _EPW // _K
_NP = 10240
_RPT = _NP // _NS


def _deg_body(row_hbm, col_hbm, degp_hbm, diagp_hbm, rowv, colv, hist, hist2):
    cid = lax.axis_index("c")
    sid = lax.axis_index("s")
    wid = cid * _NS + sid
    pltpu.sync_copy(row_hbm.at[wid], rowv)
    pltpu.sync_copy(col_hbm.at[wid], colv)
    zeros16 = jnp.zeros((16,), jnp.float32)

    def zloop(i, c):
        hist[pl.ds(i * 16, 16)] = zeros16
        hist2[pl.ds(i * 16, 16)] = zeros16
        return c

    lax.fori_loop(0, _N // 16, zloop, 0)
    ones16 = jnp.ones((16,), jnp.float32)

    def step(i, c):
        c16 = colv[pl.ds(i * 16, 16)]
        plsc.addupdate_scatter(hist, [c16], ones16)
        r16 = rowv[pl.ds(i * 16, 16)]
        plsc.addupdate_scatter(hist2, [r16], ones16, mask=r16 == c16)
        return c

    lax.fori_loop(0, _EPW // 16, step, 0)
    pltpu.sync_copy(hist, degp_hbm.at[wid])
    pltpu.sync_copy(hist2, diagp_hbm.at[wid])


_deg = pl.kernel(
    _deg_body,
    out_type=(
        jax.ShapeDtypeStruct((_NW, _N), jnp.float32),
        jax.ShapeDtypeStruct((_NW, _N), jnp.float32),
    ),
    mesh=plsc.VectorSubcoreMesh(core_axis_name="c", subcore_axis_name="s"),
    compiler_params=pltpu.CompilerParams(needs_layout_passes=False),
    scratch_types=[
        pltpu.VMEM((_EPW,), jnp.int32),
        pltpu.VMEM((_EPW,), jnp.int32),
        pltpu.VMEM((_N,), jnp.float32),
        pltpu.VMEM((_N,), jnp.float32),
    ],
)


_NB = 3


def _agg_body(z_hbm, rowp_hbm, colp_hbm, zeros_hbm, parts_hbm,
              colv, ir0, ir1, ir2, b0, b1, b2, acc,
              gA, gB, gC, sA, sB, sC, irA, irB, irC):
    irs = (ir0, ir1, ir2)
    bufs = (b0, b1, b2)
    gs = (gA, gB, gC)
    ss = (sA, sB, sC)
    irsem = (irA, irB, irC)
    cid = lax.axis_index("c")
    sid = lax.axis_index("s")
    wid = cid * _NS + sid
    pltpu.sync_copy(zeros_hbm.at[pl.ds(sid * _RPT, _RPT)],
                    acc.at[pl.ds(sid * _RPT, _RPT)])
    pltpu.sync_copy(colp_hbm.at[wid], colv)
    plsc.subcore_barrier()
    rbase = wid * _NCH * _K

    def ir_start(j, p):
        pltpu.async_copy(rowp_hbm.at[pl.ds(rbase + j * _K, _K)], irs[p],
                         irsem[p])

    def ir_wait(p):
        pltpu.make_async_copy(rowp_hbm.at[pl.ds(rbase, _K)], irs[p],
                              irsem[p]).wait()

    def g_start(j, p):
        pltpu.async_copy(z_hbm.at[colv.at[j]], bufs[p], gs[p])

    def g_wait(p):
        pltpu.make_async_copy(z_hbm.at[colv.at[0]], bufs[p], gs[p]).wait()

    def s_start(p):
        pltpu.async_copy(bufs[p], acc.at[irs[p]], ss[p], add=True)

    def s_wait(p):
        pltpu.make_async_copy(bufs[p], acc.at[irs[p]], ss[p]).wait()

    ir_start(0, 0)
    ir_start(1, 1)
    g_start(0, 0)
    g_start(1, 1)
    ng = (_NCH + 1 + 2) // 3

    def group(i, carry):
        jb = i * 3
        for b in range(3):
            j = jb + b
            pg = (b + 2) % 3

            @pl.when((j >= 1) & (j <= _NCH))
            def _():
                s_wait(pg)

            ir_start(jnp.minimum(j + 2, _NCH - 1), pg)
            g_start(jnp.minimum(j + 2, _NCH - 1), pg)
            g_wait(b)
            ir_wait(b)

            @pl.when(j < _NCH)
            def _():
                s_start(b)

        return carry

    lax.fori_loop(0, ng, group, 0)
    ir_wait(0)
    ir_wait(1)
    g_wait(0)
    g_wait(1)
    plsc.subcore_barrier()
    pltpu.sync_copy(acc.at[pl.ds(sid * _RPT, _RPT)],
                    parts_hbm.at[cid, pl.ds(sid * _RPT, _RPT)])


_agg = pl.kernel(
    _agg_body,
    out_type=jax.ShapeDtypeStruct((_NC, _NP, _D), jnp.float32),
    mesh=plsc.VectorSubcoreMesh(core_axis_name="c", subcore_axis_name="s"),
    scratch_types=(
        [pltpu.VMEM((_NCH, _K), jnp.int32)]
        + [pltpu.VMEM((_K,), jnp.int32) for _ in range(3)]
        + [pltpu.VMEM((_K, _D), jnp.float32) for _ in range(3)]
        + [pltpu.VMEM_SHARED((_NP, _D), jnp.float32)]
        + [pltpu.SemaphoreType.DMA for _ in range(9)]
    ),
)


def _prep_body(degp_ref, diagp_ref, x_ref, z_ref, dn_ref, cv_ref):
    deg = 1.0 + jnp.sum(degp_ref[...], axis=0)
    dn = 1.0 / deg
    cv = dn + _LAMB * (1.0 + jnp.sum(diagp_ref[...], axis=0))
    dn_ref[...] = dn[:, None]
    cv_ref[...] = cv[:, None]
    z_ref[...] = dn[:, None] * x_ref[...]


def _prep(degp, diagp, x):
    return pl.pallas_call(
        _prep_body,
        out_shape=(
            jax.ShapeDtypeStruct((_N, _D), jnp.float32),
            jax.ShapeDtypeStruct((_N, 1), jnp.float32),
            jax.ShapeDtypeStruct((_N, 1), jnp.float32),
        ),
    )(degp, diagp, x)


def _layer1_body(p_ref, x_ref, cv_ref, dn_ref, w_ref, b_ref, y_ref, z_ref):
    u = p_ref[0] + p_ref[1] + cv_ref[...] * x_ref[...]
    y = jnp.maximum(u @ w_ref[...] + b_ref[...][None, :], 0.0)
    y_ref[...] = y
    z_ref[...] = dn_ref[...] * y


def _layer1(p, x, cv, dn, W1, b1):
    BR = 1000
    return pl.pallas_call(
        _layer1_body,
        grid=(_N // BR,),
        in_specs=[
            pl.BlockSpec((_NC, BR, _D), lambda i: (0, i, 0)),
            pl.BlockSpec((BR, _D), lambda i: (i, 0)),
            pl.BlockSpec((BR, 1), lambda i: (i, 0)),
            pl.BlockSpec((BR, 1), lambda i: (i, 0)),
            pl.BlockSpec((_D, _D), lambda i: (0, 0)),
            pl.BlockSpec((_D,), lambda i: (0,)),
        ],
        out_specs=(
            pl.BlockSpec((BR, _D), lambda i: (i, 0)),
            pl.BlockSpec((BR, _D), lambda i: (i, 0)),
        ),
        out_shape=(
            jax.ShapeDtypeStruct((_N, _D), jnp.float32),
            jax.ShapeDtypeStruct((_N, _D), jnp.float32),
        ),
    )(p, x, cv, dn, W1, b1)


def _layer2_body(p_ref, y_ref, cv_ref, w_ref, b_ref, o_ref):
    u = p_ref[0] + p_ref[1] + cv_ref[...] * y_ref[...]
    v = u @ w_ref[...] + b_ref[...][None, :]
    m = jnp.max(v, axis=-1, keepdims=True)
    e = jnp.exp(v - m)
    o_ref[...] = v - m - jnp.log(jnp.sum(e, axis=-1, keepdims=True))


def _layer2(p, y1, cv, W2, b2):
    BR = 1000
    return pl.pallas_call(
        _layer2_body,
        grid=(_N // BR,),
        in_specs=[
            pl.BlockSpec((_NC, BR, _D), lambda i: (0, i, 0)),
            pl.BlockSpec((BR, _D), lambda i: (i, 0)),
            pl.BlockSpec((BR, 1), lambda i: (i, 0)),
            pl.BlockSpec((_D, _D), lambda i: (0, 0)),
            pl.BlockSpec((_D,), lambda i: (0,)),
        ],
        out_specs=pl.BlockSpec((BR, _D), lambda i: (i, 0)),
        out_shape=jax.ShapeDtypeStruct((_N, _D), jnp.float32),
    )(p, y1, cv, W2, b2)


def kernel(x, edge_index, W1, b1, W2, b2):
    row = edge_index[0].astype(jnp.int32)
    col = edge_index[1].astype(jnp.int32)
    rowf = row.reshape(_NW, _EPW)
    colf = col.reshape(_NW, _EPW)
    colp = col.reshape(_NW, _NCH, _K)
    zeros = jnp.zeros((_NP, _D), jnp.float32)

    degp, diagp = _deg(rowf, colf)
    z1, dn, cv = _prep(degp, diagp, x)
    p1 = _agg(z1, row, colp, zeros)
    y1, z2 = _layer1(p1[:, :_N], x, cv, dn, W1, b1)
    p2 = _agg(z2, row, colp, zeros)
    return _layer2(p2[:, :_N], y1, cv, W2, b2)


# speedup vs baseline: 3.1532x; 1.0587x over previous
import jax
import jax.numpy as jnp
from jax import lax
from jax.experimental import pallas as pl
from jax.experimental.pallas import tpu as pltpu
from jax.experimental.pallas import tpu_sc as plsc

_LAMB = 1.0
_N = 10000
_E = 320000
_D = 128
_NC = 2
_NS = 16
_NW = _NC * _NS
_EPW = _E // _NW
_K = 80
_NCH = _EPW // _K
_NP = 10240
_RPT = _NP // _NS


def _deg_body(row_hbm, col_hbm, degp_hbm, diagp_hbm, rowv, colv, hist, hist2):
    cid = lax.axis_index("c")
    sid = lax.axis_index("s")
    wid = cid * _NS + sid
    pltpu.sync_copy(row_hbm.at[pl.ds(wid * _EPW, _EPW)], rowv)
    pltpu.sync_copy(col_hbm.at[pl.ds(wid * _EPW, _EPW)], colv)
    zeros16 = jnp.zeros((16,), jnp.float32)

    def zloop(i, c):
        hist[pl.ds(i * 16, 16)] = zeros16
        hist2[pl.ds(i * 16, 16)] = zeros16
        return c

    lax.fori_loop(0, _N // 16, zloop, 0)
    ones16 = jnp.ones((16,), jnp.float32)

    def step(i, c):
        c16 = colv[pl.ds(i * 16, 16)]
        plsc.addupdate_scatter(hist, [c16], ones16)
        r16 = rowv[pl.ds(i * 16, 16)]
        plsc.addupdate_scatter(hist2, [r16], ones16, mask=r16 == c16)
        return c

    lax.fori_loop(0, _EPW // 16, step, 0)
    pltpu.sync_copy(hist, degp_hbm.at[wid])
    pltpu.sync_copy(hist2, diagp_hbm.at[wid])


_deg = pl.kernel(
    _deg_body,
    out_type=(
        jax.ShapeDtypeStruct((_NW, _N), jnp.float32),
        jax.ShapeDtypeStruct((_NW, _N), jnp.float32),
    ),
    mesh=plsc.VectorSubcoreMesh(core_axis_name="c", subcore_axis_name="s"),
    compiler_params=pltpu.CompilerParams(needs_layout_passes=False),
    scratch_types=[
        pltpu.VMEM((_EPW,), jnp.int32),
        pltpu.VMEM((_EPW,), jnp.int32),
        pltpu.VMEM((_N,), jnp.float32),
        pltpu.VMEM((_N,), jnp.float32),
    ],
)


_NB = 3


def _agg_body(z_hbm, rowp_hbm, colp_hbm, zeros_hbm, parts_hbm,
              colv, ir0, ir1, ir2, b0, b1, b2, acc,
              gA, gB, gC, sA, sB, sC, irA, irB, irC):
    irs = (ir0, ir1, ir2)
    bufs = (b0, b1, b2)
    gs = (gA, gB, gC)
    ss = (sA, sB, sC)
    irsem = (irA, irB, irC)
    cid = lax.axis_index("c")
    sid = lax.axis_index("s")
    wid = cid * _NS + sid
    pltpu.sync_copy(zeros_hbm.at[pl.ds(sid * _RPT, _RPT)],
                    acc.at[pl.ds(sid * _RPT, _RPT)])
    pltpu.sync_copy(colp_hbm.at[pl.ds(wid * _EPW, _EPW)], colv)
    plsc.subcore_barrier()
    rbase = wid * _EPW

    def ir_start(j, p):
        pltpu.async_copy(rowp_hbm.at[pl.ds(rbase + j * _K, _K)], irs[p],
                         irsem[p])

    def ir_wait(p):
        pltpu.make_async_copy(rowp_hbm.at[pl.ds(rbase, _K)], irs[p],
                              irsem[p]).wait()

    def g_start(j, p):
        pltpu.async_copy(z_hbm.at[colv.at[pl.ds(j * _K, _K)]], bufs[p], gs[p])

    def g_wait(p):
        pltpu.make_async_copy(z_hbm.at[colv.at[pl.ds(0, _K)]], bufs[p],
                              gs[p]).wait()

    def s_start(p):
        pltpu.async_copy(bufs[p], acc.at[irs[p]], ss[p], add=True)

    def s_wait(p):
        pltpu.make_async_copy(bufs[p], acc.at[irs[p]], ss[p]).wait()

    ir_start(0, 0)
    ir_start(1, 1)
    g_start(0, 0)
    g_start(1, 1)
    ng = (_NCH + 1 + 2) // 3

    def group(i, carry):
        jb = i * 3
        for b in range(3):
            j = jb + b
            pg = (b + 2) % 3

            @pl.when((j >= 1) & (j <= _NCH))
            def _():
                s_wait(pg)

            ir_start(jnp.minimum(j + 2, _NCH - 1), pg)
            g_start(jnp.minimum(j + 2, _NCH - 1), pg)
            g_wait(b)
            ir_wait(b)

            @pl.when(j < _NCH)
            def _():
                s_start(b)

        return carry

    lax.fori_loop(0, ng, group, 0)
    ir_wait(0)
    ir_wait(1)
    g_wait(0)
    g_wait(1)
    plsc.subcore_barrier()
    pltpu.sync_copy(acc.at[pl.ds(sid * _RPT, _RPT)],
                    parts_hbm.at[cid, pl.ds(sid * _RPT, _RPT)])


_agg = pl.kernel(
    _agg_body,
    out_type=jax.ShapeDtypeStruct((_NC, _NP, _D), jnp.float32),
    mesh=plsc.VectorSubcoreMesh(core_axis_name="c", subcore_axis_name="s"),
    scratch_types=(
        [pltpu.VMEM((_EPW,), jnp.int32)]
        + [pltpu.VMEM((_K,), jnp.int32) for _ in range(3)]
        + [pltpu.VMEM((_K, _D), jnp.float32) for _ in range(3)]
        + [pltpu.VMEM_SHARED((_NP, _D), jnp.float32)]
        + [pltpu.SemaphoreType.DMA for _ in range(9)]
    ),
)


def _prep_body(degp_ref, diagp_ref, x_ref, z_ref, dn_ref, cv_ref):
    deg = 1.0 + jnp.sum(degp_ref[...], axis=0)
    dn = 1.0 / deg
    cv = dn + _LAMB * (1.0 + jnp.sum(diagp_ref[...], axis=0))
    dn_ref[...] = dn[:, None]
    cv_ref[...] = cv[:, None]
    z_ref[...] = dn[:, None] * x_ref[...]


def _prep(degp, diagp, x):
    return pl.pallas_call(
        _prep_body,
        out_shape=(
            jax.ShapeDtypeStruct((_N, _D), jnp.float32),
            jax.ShapeDtypeStruct((_N, 1), jnp.float32),
            jax.ShapeDtypeStruct((_N, 1), jnp.float32),
        ),
    )(degp, diagp, x)


def _layer1_body(p_ref, x_ref, cv_ref, dn_ref, w_ref, b_ref, y_ref, z_ref):
    u = p_ref[0] + p_ref[1] + cv_ref[...] * x_ref[...]
    y = jnp.maximum(u @ w_ref[...] + b_ref[...][None, :], 0.0)
    y_ref[...] = y
    z_ref[...] = dn_ref[...] * y


def _layer1(p, x, cv, dn, W1, b1):
    BR = 1000
    return pl.pallas_call(
        _layer1_body,
        grid=(_N // BR,),
        in_specs=[
            pl.BlockSpec((_NC, BR, _D), lambda i: (0, i, 0)),
            pl.BlockSpec((BR, _D), lambda i: (i, 0)),
            pl.BlockSpec((BR, 1), lambda i: (i, 0)),
            pl.BlockSpec((BR, 1), lambda i: (i, 0)),
            pl.BlockSpec((_D, _D), lambda i: (0, 0)),
            pl.BlockSpec((_D,), lambda i: (0,)),
        ],
        out_specs=(
            pl.BlockSpec((BR, _D), lambda i: (i, 0)),
            pl.BlockSpec((BR, _D), lambda i: (i, 0)),
        ),
        out_shape=(
            jax.ShapeDtypeStruct((_N, _D), jnp.float32),
            jax.ShapeDtypeStruct((_N, _D), jnp.float32),
        ),
    )(p, x, cv, dn, W1, b1)


def _layer2_body(p_ref, y_ref, cv_ref, w_ref, b_ref, o_ref):
    u = p_ref[0] + p_ref[1] + cv_ref[...] * y_ref[...]
    v = u @ w_ref[...] + b_ref[...][None, :]
    m = jnp.max(v, axis=-1, keepdims=True)
    e = jnp.exp(v - m)
    o_ref[...] = v - m - jnp.log(jnp.sum(e, axis=-1, keepdims=True))


def _layer2(p, y1, cv, W2, b2):
    BR = 1000
    return pl.pallas_call(
        _layer2_body,
        grid=(_N // BR,),
        in_specs=[
            pl.BlockSpec((_NC, BR, _D), lambda i: (0, i, 0)),
            pl.BlockSpec((BR, _D), lambda i: (i, 0)),
            pl.BlockSpec((BR, 1), lambda i: (i, 0)),
            pl.BlockSpec((_D, _D), lambda i: (0, 0)),
            pl.BlockSpec((_D,), lambda i: (0,)),
        ],
        out_specs=pl.BlockSpec((BR, _D), lambda i: (i, 0)),
        out_shape=jax.ShapeDtypeStruct((_N, _D), jnp.float32),
    )(p, y1, cv, W2, b2)


def kernel(x, edge_index, W1, b1, W2, b2):
    row = edge_index[0].astype(jnp.int32)
    col = edge_index[1].astype(jnp.int32)
    zeros = jnp.zeros((_NP, _D), jnp.float32)

    degp, diagp = _deg(row, col)
    z1, dn, cv = _prep(degp, diagp, x)
    p1 = _agg(z1, row, col, zeros)
    y1, z2 = _layer1(p1, x, cv, dn, W1, b1)
    p2 = _agg(z2, row, col, zeros)
    return _layer2(p2, y1, cv, W2, b2)
